# baseline (speedup 1.0000x reference)
import numpy as np
import jax
import jax.numpy as jnp
from jax import lax
from jax.experimental import pallas as pl
from jax.experimental.pallas import tpu as pltpu

_B = 32
_D = 2048
_NF = 32768
_MAXK = 64
_TN = 1024
_NT = _NF // _TN
_CH = 8192
_NCH = _NF // _CH
_TREE_BITS = 5


def _topk_mask_body(x_ref, w_ref, b_ref, k_ref, out_ref, skey_ref, skey16_ref,
                    tree_ref):
    _MININT = jnp.int32(-2147483648)
    _LOW31 = jnp.int32(2147483647)
    i = pl.program_id(0)

    @pl.when(i == 0)
    def _init():
        tree_ref[...] = jnp.zeros((_B, 32), jnp.int32)

    z = jnp.dot(x_ref[...], w_ref[...], preferred_element_type=jnp.float32)
    z = z + b_ref[...]
    out_ref[:, pl.ds(i * _TN, _TN)] = z
    bits = lax.bitcast_convert_type(z, jnp.int32)
    skey = bits ^ (lax.shift_right_arithmetic(bits, 31) & _LOW31)
    skey_ref[:, pl.ds(i * _TN, _TN)] = skey
    skey16_ref[:, pl.ds(i * _TN, _TN)] = (
        lax.shift_right_arithmetic(skey, 16).astype(jnp.int16))
    bucket = lax.shift_right_logical(skey ^ _MININT, 32 - _TREE_BITS)
    cnts = [
        jnp.sum((bucket >= jnp.int32(m)).astype(jnp.int32), axis=1,
                keepdims=True)
        for m in range(32)
    ]
    tree_ref[...] += jnp.concatenate(cnts, axis=1)

    @pl.when(i == _NT - 1)
    def _finalize():
        kk = jnp.minimum(k_ref[0], _MAXK).astype(jnp.int32)
        zeros = jnp.zeros((_B, 1), jnp.int32)

        def _count_ge(ts):
            def cb(c, acc):
                s = skey_ref[:, pl.ds(c * _CH, _CH)]
                return acc + (s >= ts).astype(jnp.int32)
            acc = lax.fori_loop(0, _NCH, cb,
                                jnp.zeros((_B, _CH), jnp.int32))
            return jnp.sum(acc, axis=1, keepdims=True)

        lane = lax.broadcasted_iota(jnp.int32, (_B, 32), 1)
        tree = tree_ref[...]
        t_u = zeros
        for j in range(_TREE_BITS):
            cand_u = t_u | jnp.int32(np.uint32(1 << (31 - j)).view(np.int32))
            m = lax.shift_right_logical(cand_u, 32 - _TREE_BITS)
            cnt = jnp.sum(jnp.where(lane == m, tree, 0), axis=1,
                          keepdims=True)
            t_u = jnp.where(cnt >= kk, cand_u, t_u)

        def _bit(j, t_u):
            bitv = lax.shift_left(jnp.int32(1), 31 - j)
            cand_u = t_u | bitv
            cnt = _count_ge(cand_u ^ _MININT)
            return jnp.where(cnt >= kk, cand_u, t_u)

        def _bit16(j, t_u):
            bitv = lax.shift_left(jnp.int32(1), 31 - j)
            cand_u = t_u | bitv
            c16 = (lax.shift_right_logical(cand_u, 16)
                   ^ jnp.int32(0x8000)).astype(jnp.int16)

            def cb(c, acc):
                s = skey16_ref[:, pl.ds(c * _CH, _CH)]
                return acc + (s >= c16).astype(jnp.int16)

            acc = lax.fori_loop(0, _NCH, cb,
                                jnp.zeros((_B, _CH), jnp.int16))
            h = _CH
            while h > 512:
                h //= 2
                acc = acc[:, :h] + acc[:, h:2 * h]
            cnt = jnp.sum(acc.astype(jnp.int32), axis=1, keepdims=True)
            return jnp.where(cnt >= kk, cand_u, t_u)

        t_u16 = lax.fori_loop(_TREE_BITS, 16, _bit16, t_u)
        t_u24 = lax.fori_loop(16, 24, _bit, t_u16)

        def _cols(c):
            return c * _CH + lax.broadcasted_iota(jnp.int32, (_B, _CH), 1)

        def _writeback(t_s, c_idx):
            bits_t = jnp.where(t_s >= 0, t_s, t_s ^ _LOW31)
            t_f = lax.bitcast_convert_type(bits_t, jnp.float32)

            def cb(c, cnt):
                zc = out_ref[:, pl.ds(c * _CH, _CH)]
                keep = (zc > t_f) | ((zc == t_f) & (_cols(c) <= c_idx))
                out_ref[:, pl.ds(c * _CH, _CH)] = jnp.where(
                    keep, jnp.maximum(zc, 0.0), 0.0)
                return cnt + jnp.sum(keep.astype(jnp.int32), axis=1,
                                     keepdims=True)
            return lax.fori_loop(0, _NCH, cb, zeros)

        far = jnp.full((_B, 1), _NF - 1, jnp.int32)
        cnt_kept = _writeback(t_u24 ^ _MININT, far)

        @pl.when(jnp.any(cnt_kept != kk))
        def _slow():
            t_u = lax.fori_loop(24, 32, _bit, t_u24)
            t_s = t_u ^ _MININT

            def gtb(c, cnt):
                s = skey_ref[:, pl.ds(c * _CH, _CH)]
                return cnt + jnp.sum((s > t_s).astype(jnp.int32), axis=1,
                                     keepdims=True)

            cnt_gt = lax.fori_loop(0, _NCH, gtb, zeros)
            r = kk - cnt_gt

            def cnt_eq_le(m):
                def cb(c, cnt):
                    s = skey_ref[:, pl.ds(c * _CH, _CH)]
                    hit = (s == t_s) & (_cols(c) <= m)
                    return cnt + jnp.sum(hit.astype(jnp.int32), axis=1,
                                         keepdims=True)
                return lax.fori_loop(0, _NCH, cb, zeros)

            def bs(j, lohi):
                lo, hi = lohi
                mid = lax.shift_right_arithmetic(lo + hi, 1)
                ok = cnt_eq_le(mid) >= r
                return (jnp.where(ok, lo, mid + 1), jnp.where(ok, mid, hi))

            _, hi = lax.fori_loop(0, 15, bs, (zeros, far))
            _writeback(t_s, hi)


def kernel(x, W_enc, b_enc, k):
    b2 = jnp.reshape(b_enc.astype(jnp.float32), (1, _NF))
    karr = jnp.reshape(jnp.asarray(k, jnp.int32), (1,))
    return pl.pallas_call(
        _topk_mask_body,
        grid=(_NT,),
        in_specs=[
            pl.BlockSpec((_B, _D), lambda i: (0, 0)),
            pl.BlockSpec((_D, _TN), lambda i: (0, i)),
            pl.BlockSpec((1, _TN), lambda i: (0, i)),
            pl.BlockSpec(memory_space=pltpu.SMEM),
        ],
        out_specs=pl.BlockSpec((_B, _NF), lambda i: (0, 0)),
        out_shape=jax.ShapeDtypeStruct((_B, _NF), jnp.float32),
        scratch_shapes=[pltpu.VMEM((_B, _NF), jnp.int32),
                        pltpu.VMEM((_B, _NF), jnp.int16),
                        pltpu.VMEM((_B, 32), jnp.int32)],
        compiler_params=pltpu.CompilerParams(
            dimension_semantics=("arbitrary",)),
    )(x.astype(jnp.float32), W_enc.astype(jnp.float32), b2, karr)


# speedup vs baseline: 1.0186x; 1.0186x over previous
import numpy as np
import jax
import jax.numpy as jnp
from jax import lax
from jax.experimental import pallas as pl
from jax.experimental.pallas import tpu as pltpu

_B = 32
_D = 2048
_NF = 32768
_MAXK = 64
_TN = 1024
_NT = _NF // _TN
_CH = 4096
_NCH = _NF // _CH
_TREE_BITS = 5


def _topk_mask_body(x_ref, w_ref, b_ref, k_ref, out_ref, skey_ref, skey16_ref,
                    tree_ref):
    _MININT = jnp.int32(-2147483648)
    _LOW31 = jnp.int32(2147483647)
    i = pl.program_id(0)

    @pl.when(i == 0)
    def _init():
        tree_ref[...] = jnp.zeros((_B, 32), jnp.int32)

    z = jnp.dot(x_ref[...], w_ref[...], preferred_element_type=jnp.float32)
    z = z + b_ref[...]
    out_ref[:, pl.ds(i * _TN, _TN)] = z
    bits = lax.bitcast_convert_type(z, jnp.int32)
    skey = bits ^ (lax.shift_right_arithmetic(bits, 31) & _LOW31)
    skey_ref[:, pl.ds(i * _TN, _TN)] = skey
    skey16_ref[:, pl.ds(i * _TN, _TN)] = (
        lax.shift_right_arithmetic(skey, 16).astype(jnp.int16))
    bucket = lax.shift_right_logical(skey ^ _MININT, 32 - _TREE_BITS)
    cnts = [
        jnp.sum((bucket >= jnp.int32(m)).astype(jnp.int32), axis=1,
                keepdims=True)
        for m in range(32)
    ]
    tree_ref[...] += jnp.concatenate(cnts, axis=1)

    @pl.when(i == _NT - 1)
    def _finalize():
        kk = jnp.minimum(k_ref[0], _MAXK).astype(jnp.int32)
        zeros = jnp.zeros((_B, 1), jnp.int32)

        def _count_ge(ts):
            def cb(c, acc):
                s = skey_ref[:, pl.ds(c * _CH, _CH)]
                return acc + (s >= ts).astype(jnp.int32)
            acc = lax.fori_loop(0, _NCH, cb,
                                jnp.zeros((_B, _CH), jnp.int32))
            return jnp.sum(acc, axis=1, keepdims=True)

        lane = lax.broadcasted_iota(jnp.int32, (_B, 32), 1)
        tree = tree_ref[...]
        t_u = zeros
        for j in range(_TREE_BITS):
            cand_u = t_u | jnp.int32(np.uint32(1 << (31 - j)).view(np.int32))
            m = lax.shift_right_logical(cand_u, 32 - _TREE_BITS)
            cnt = jnp.sum(jnp.where(lane == m, tree, 0), axis=1,
                          keepdims=True)
            t_u = jnp.where(cnt >= kk, cand_u, t_u)

        def _bit(j, t_u):
            bitv = lax.shift_left(jnp.int32(1), 31 - j)
            cand_u = t_u | bitv
            cnt = _count_ge(cand_u ^ _MININT)
            return jnp.where(cnt >= kk, cand_u, t_u)

        def _bit16(j, t_u):
            bitv = lax.shift_left(jnp.int32(1), 31 - j)
            cand_u = t_u | bitv
            c16 = (lax.shift_right_logical(cand_u, 16)
                   ^ jnp.int32(0x8000)).astype(jnp.int16)

            def cb(c, acc):
                s = skey16_ref[:, pl.ds(c * _CH, _CH)]
                return acc + (s >= c16).astype(jnp.int16)

            acc = lax.fori_loop(0, _NCH, cb,
                                jnp.zeros((_B, _CH), jnp.int16))
            h = _CH
            while h > 512:
                h //= 2
                acc = acc[:, :h] + acc[:, h:2 * h]
            cnt = jnp.sum(acc.astype(jnp.int32), axis=1, keepdims=True)
            return jnp.where(cnt >= kk, cand_u, t_u)

        t_u16 = lax.fori_loop(_TREE_BITS, 16, _bit16, t_u)
        t_u24 = lax.fori_loop(16, 24, _bit, t_u16)

        def _cols(c):
            return c * _CH + lax.broadcasted_iota(jnp.int32, (_B, _CH), 1)

        def _writeback(t_s, c_idx):
            bits_t = jnp.where(t_s >= 0, t_s, t_s ^ _LOW31)
            t_f = lax.bitcast_convert_type(bits_t, jnp.float32)

            def cb(c, cnt):
                zc = out_ref[:, pl.ds(c * _CH, _CH)]
                keep = (zc > t_f) | ((zc == t_f) & (_cols(c) <= c_idx))
                out_ref[:, pl.ds(c * _CH, _CH)] = jnp.where(
                    keep, jnp.maximum(zc, 0.0), 0.0)
                return cnt + jnp.sum(keep.astype(jnp.int32), axis=1,
                                     keepdims=True)
            return lax.fori_loop(0, _NCH, cb, zeros)

        far = jnp.full((_B, 1), _NF - 1, jnp.int32)
        cnt_kept = _writeback(t_u24 ^ _MININT, far)

        @pl.when(jnp.any(cnt_kept != kk))
        def _slow():
            t_u = lax.fori_loop(24, 32, _bit, t_u24)
            t_s = t_u ^ _MININT

            def gtb(c, cnt):
                s = skey_ref[:, pl.ds(c * _CH, _CH)]
                return cnt + jnp.sum((s > t_s).astype(jnp.int32), axis=1,
                                     keepdims=True)

            cnt_gt = lax.fori_loop(0, _NCH, gtb, zeros)
            r = kk - cnt_gt

            def cnt_eq_le(m):
                def cb(c, cnt):
                    s = skey_ref[:, pl.ds(c * _CH, _CH)]
                    hit = (s == t_s) & (_cols(c) <= m)
                    return cnt + jnp.sum(hit.astype(jnp.int32), axis=1,
                                         keepdims=True)
                return lax.fori_loop(0, _NCH, cb, zeros)

            def bs(j, lohi):
                lo, hi = lohi
                mid = lax.shift_right_arithmetic(lo + hi, 1)
                ok = cnt_eq_le(mid) >= r
                return (jnp.where(ok, lo, mid + 1), jnp.where(ok, mid, hi))

            _, hi = lax.fori_loop(0, 15, bs, (zeros, far))
            _writeback(t_s, hi)


def kernel(x, W_enc, b_enc, k):
    b2 = jnp.reshape(b_enc.astype(jnp.float32), (1, _NF))
    karr = jnp.reshape(jnp.asarray(k, jnp.int32), (1,))
    return pl.pallas_call(
        _topk_mask_body,
        grid=(_NT,),
        in_specs=[
            pl.BlockSpec((_B, _D), lambda i: (0, 0)),
            pl.BlockSpec((_D, _TN), lambda i: (0, i)),
            pl.BlockSpec((1, _TN), lambda i: (0, i)),
            pl.BlockSpec(memory_space=pltpu.SMEM),
        ],
        out_specs=pl.BlockSpec((_B, _NF), lambda i: (0, 0)),
        out_shape=jax.ShapeDtypeStruct((_B, _NF), jnp.float32),
        scratch_shapes=[pltpu.VMEM((_B, _NF), jnp.int32),
                        pltpu.VMEM((_B, _NF), jnp.int16),
                        pltpu.VMEM((_B, 32), jnp.int32)],
        compiler_params=pltpu.CompilerParams(
            dimension_semantics=("arbitrary",)),
    )(x.astype(jnp.float32), W_enc.astype(jnp.float32), b2, karr)


# speedup vs baseline: 1.0346x; 1.0158x over previous
import numpy as np
import jax
import jax.numpy as jnp
from jax import lax
from jax.experimental import pallas as pl
from jax.experimental.pallas import tpu as pltpu

_B = 32
_D = 2048
_NF = 32768
_MAXK = 64
_TN = 2048
_NT = _NF // _TN
_CH = 4096
_NCH = _NF // _CH
_TREE_BITS = 5


def _topk_mask_body(x_ref, w_ref, b_ref, k_ref, out_ref, skey_ref, skey16_ref,
                    tree_ref):
    _MININT = jnp.int32(-2147483648)
    _LOW31 = jnp.int32(2147483647)
    i = pl.program_id(0)

    @pl.when(i == 0)
    def _init():
        tree_ref[...] = jnp.zeros((_B, 32), jnp.int32)

    z = jnp.dot(x_ref[...], w_ref[...], preferred_element_type=jnp.float32)
    z = z + b_ref[...]
    out_ref[:, pl.ds(i * _TN, _TN)] = z
    bits = lax.bitcast_convert_type(z, jnp.int32)
    skey = bits ^ (lax.shift_right_arithmetic(bits, 31) & _LOW31)
    skey_ref[:, pl.ds(i * _TN, _TN)] = skey
    skey16_ref[:, pl.ds(i * _TN, _TN)] = (
        lax.shift_right_arithmetic(skey, 16).astype(jnp.int16))
    bucket = lax.shift_right_logical(skey ^ _MININT, 32 - _TREE_BITS)
    cnts = [
        jnp.sum((bucket >= jnp.int32(m)).astype(jnp.int32), axis=1,
                keepdims=True)
        for m in range(32)
    ]
    tree_ref[...] += jnp.concatenate(cnts, axis=1)

    @pl.when(i == _NT - 1)
    def _finalize():
        kk = jnp.minimum(k_ref[0], _MAXK).astype(jnp.int32)
        zeros = jnp.zeros((_B, 1), jnp.int32)

        def _count_ge(ts):
            def cb(c, acc):
                s = skey_ref[:, pl.ds(c * _CH, _CH)]
                return acc + (s >= ts).astype(jnp.int32)
            acc = lax.fori_loop(0, _NCH, cb,
                                jnp.zeros((_B, _CH), jnp.int32))
            return jnp.sum(acc, axis=1, keepdims=True)

        lane = lax.broadcasted_iota(jnp.int32, (_B, 32), 1)
        tree = tree_ref[...]
        t_u = zeros
        for j in range(_TREE_BITS):
            cand_u = t_u | jnp.int32(np.uint32(1 << (31 - j)).view(np.int32))
            m = lax.shift_right_logical(cand_u, 32 - _TREE_BITS)
            cnt = jnp.sum(jnp.where(lane == m, tree, 0), axis=1,
                          keepdims=True)
            t_u = jnp.where(cnt >= kk, cand_u, t_u)

        def _bit(j, t_u):
            bitv = lax.shift_left(jnp.int32(1), 31 - j)
            cand_u = t_u | bitv
            cnt = _count_ge(cand_u ^ _MININT)
            return jnp.where(cnt >= kk, cand_u, t_u)

        def _bit16(j, t_u):
            bitv = lax.shift_left(jnp.int32(1), 31 - j)
            cand_u = t_u | bitv
            c16 = (lax.shift_right_logical(cand_u, 16)
                   ^ jnp.int32(0x8000)).astype(jnp.int16)

            def cb(c, acc):
                s = skey16_ref[:, pl.ds(c * _CH, _CH)]
                return acc + (s >= c16).astype(jnp.int16)

            acc = lax.fori_loop(0, _NCH, cb,
                                jnp.zeros((_B, _CH), jnp.int16))
            h = _CH
            while h > 512:
                h //= 2
                acc = acc[:, :h] + acc[:, h:2 * h]
            cnt = jnp.sum(acc.astype(jnp.int32), axis=1, keepdims=True)
            return jnp.where(cnt >= kk, cand_u, t_u)

        t_u16 = lax.fori_loop(_TREE_BITS, 16, _bit16, t_u)
        t_u24 = lax.fori_loop(16, 24, _bit, t_u16)

        def _cols(c):
            return c * _CH + lax.broadcasted_iota(jnp.int32, (_B, _CH), 1)

        def _writeback(t_s, c_idx):
            bits_t = jnp.where(t_s >= 0, t_s, t_s ^ _LOW31)
            t_f = lax.bitcast_convert_type(bits_t, jnp.float32)

            def cb(c, cnt):
                zc = out_ref[:, pl.ds(c * _CH, _CH)]
                keep = (zc > t_f) | ((zc == t_f) & (_cols(c) <= c_idx))
                out_ref[:, pl.ds(c * _CH, _CH)] = jnp.where(
                    keep, jnp.maximum(zc, 0.0), 0.0)
                return cnt + jnp.sum(keep.astype(jnp.int32), axis=1,
                                     keepdims=True)
            return lax.fori_loop(0, _NCH, cb, zeros)

        far = jnp.full((_B, 1), _NF - 1, jnp.int32)
        cnt_kept = _writeback(t_u24 ^ _MININT, far)

        @pl.when(jnp.any(cnt_kept != kk))
        def _slow():
            t_u = lax.fori_loop(24, 32, _bit, t_u24)
            t_s = t_u ^ _MININT

            def gtb(c, cnt):
                s = skey_ref[:, pl.ds(c * _CH, _CH)]
                return cnt + jnp.sum((s > t_s).astype(jnp.int32), axis=1,
                                     keepdims=True)

            cnt_gt = lax.fori_loop(0, _NCH, gtb, zeros)
            r = kk - cnt_gt

            def cnt_eq_le(m):
                def cb(c, cnt):
                    s = skey_ref[:, pl.ds(c * _CH, _CH)]
                    hit = (s == t_s) & (_cols(c) <= m)
                    return cnt + jnp.sum(hit.astype(jnp.int32), axis=1,
                                         keepdims=True)
                return lax.fori_loop(0, _NCH, cb, zeros)

            def bs(j, lohi):
                lo, hi = lohi
                mid = lax.shift_right_arithmetic(lo + hi, 1)
                ok = cnt_eq_le(mid) >= r
                return (jnp.where(ok, lo, mid + 1), jnp.where(ok, mid, hi))

            _, hi = lax.fori_loop(0, 15, bs, (zeros, far))
            _writeback(t_s, hi)


def kernel(x, W_enc, b_enc, k):
    b2 = jnp.reshape(b_enc.astype(jnp.float32), (1, _NF))
    karr = jnp.reshape(jnp.asarray(k, jnp.int32), (1,))
    return pl.pallas_call(
        _topk_mask_body,
        grid=(_NT,),
        in_specs=[
            pl.BlockSpec((_B, _D), lambda i: (0, 0)),
            pl.BlockSpec((_D, _TN), lambda i: (0, i)),
            pl.BlockSpec((1, _TN), lambda i: (0, i)),
            pl.BlockSpec(memory_space=pltpu.SMEM),
        ],
        out_specs=pl.BlockSpec((_B, _NF), lambda i: (0, 0)),
        out_shape=jax.ShapeDtypeStruct((_B, _NF), jnp.float32),
        scratch_shapes=[pltpu.VMEM((_B, _NF), jnp.int32),
                        pltpu.VMEM((_B, _NF), jnp.int16),
                        pltpu.VMEM((_B, 32), jnp.int32)],
        compiler_params=pltpu.CompilerParams(
            dimension_semantics=("arbitrary",)),
    )(x.astype(jnp.float32), W_enc.astype(jnp.float32), b2, karr)
